# baseline (speedup 1.0000x reference)
import functools

import jax
import jax.numpy as jnp
from jax import lax
from jax.experimental import pallas as pl
from jax.experimental.pallas import tpu as pltpu
from jax.experimental.pallas import tpu_sc as plsc

_VOCAB = 1000000
_EMB = 64
_B = 16384
_L = 200

_TBLK = 24576

_NW = 32
_RPW = _B // _NW
_CH = 128
_NCH = _RPW // _CH
_IDXN = _CH * _L


def _tv_body(t_ref, w_ref, o_ref):
    o_ref[...] = jnp.sum(t_ref[...] * w_ref[...], axis=0)


def _table_times_w(table_t, W):
    return pl.pallas_call(
        _tv_body,
        grid=(pl.cdiv(_VOCAB, _TBLK),),
        in_specs=[
            pl.BlockSpec((_EMB, _TBLK), lambda i: (0, i)),
            pl.BlockSpec((_EMB, 1), lambda i: (0, 0)),
        ],
        out_specs=pl.BlockSpec((_TBLK,), lambda i: (i,)),
        out_shape=jax.ShapeDtypeStruct((_VOCAB,), jnp.float32),
    )(table_t, W)


def _sc_pool(tv, xflat, length, b16):
    mesh = plsc.VectorSubcoreMesh(core_axis_name="c", subcore_axis_name="s")

    @functools.partial(
        pl.kernel,
        mesh=mesh,
        out_type=jax.ShapeDtypeStruct((_B,), jnp.float32),
        scratch_types=[
            pltpu.VMEM((_IDXN // 128, 128), jnp.int32),
            pltpu.VMEM((_IDXN,), jnp.float32),
            pltpu.VMEM((_RPW,), jnp.float32),
            pltpu.VMEM((_RPW,), jnp.float32),
            pltpu.VMEM((16,), jnp.float32),
            pltpu.SemaphoreType.DMA,
        ],
    )
    def sck(tv_hbm, x_hbm, len_hbm, b_hbm, out_hbm,
            idx_v, val_v, len_v, out_v, b_v, sem):
        w = lax.axis_index("s") * 2 + lax.axis_index("c")
        row0 = w * _RPW
        pltpu.sync_copy(len_hbm.at[pl.ds(row0, _RPW)], len_v)
        pltpu.sync_copy(b_hbm, b_v)
        bias = b_v[...]
        xrow0 = w * (_RPW * _L // 128)
        for c in range(_NCH):
            pltpu.sync_copy(
                x_hbm.at[pl.ds(xrow0 + c * (_IDXN // 128), _IDXN // 128), :],
                idx_v)

            def gbody(j8, _):
                hs = [pltpu.async_copy(
                          tv_hbm.at[idx_v.at[j8 * 8 + k]],
                          val_v.at[pl.ds((j8 * 8 + k) * 128, 128)], sem)
                      for k in range(8)]
                for h in hs:
                    h.wait()
                return 0

            lax.fori_loop(0, _IDXN // 128 // 8, gbody, 0)
            for g in range(_CH // 16):
                goff = g * 16 * _L

                def jbody(j, acc, _goff=goff):
                    acc = acc + val_v[pl.ds(_goff + j * 16, 16)]
                    return acc

                acc = lax.fori_loop(0, _L, jbody,
                                    jnp.zeros((16,), jnp.float32))
                o16 = c * _CH + g * 16
                out_v[pl.ds(o16, 16)] = acc / len_v[pl.ds(o16, 16)] + bias
        pltpu.sync_copy(out_v, out_hbm.at[pl.ds(row0, _RPW)])

    return sck(tv, xflat, length, b16)


def kernel(x, length, table, W, b):
    tv = _table_times_w(table.T, W)
    xflat = (x.reshape(_B // 16, 16, _L).transpose(0, 2, 1)
             .reshape(_B * _L // 128, 128))
    b16 = jnp.broadcast_to(b.astype(jnp.float32), (16,))
    rows = _sc_pool(tv, xflat, length, b16)
    return rows.reshape(_B, 1)


# speedup vs baseline: 1.6520x; 1.6520x over previous
import functools

import jax
import jax.numpy as jnp
from jax import lax
from jax.experimental import pallas as pl
from jax.experimental.pallas import tpu as pltpu
from jax.experimental.pallas import tpu_sc as plsc

_VOCAB = 1000000
_EMB = 64
_B = 16384
_L = 200

_TBLK = 24576

_NW = 32
_RPW = _B // _NW
_CH = 128
_NCH = _RPW // _CH
_IDXN = _CH * _L


def _tv_body(t_ref, w_ref, o_ref):
    o_ref[...] = jnp.sum(t_ref[...] * w_ref[...], axis=0)


def _table_times_w(table_t, W):
    return pl.pallas_call(
        _tv_body,
        grid=(pl.cdiv(_VOCAB, _TBLK),),
        in_specs=[
            pl.BlockSpec((_EMB, _TBLK), lambda i: (0, i)),
            pl.BlockSpec((_EMB, 1), lambda i: (0, 0)),
        ],
        out_specs=pl.BlockSpec((_TBLK,), lambda i: (i,)),
        out_shape=jax.ShapeDtypeStruct((_VOCAB,), jnp.float32),
    )(table_t, W)


def _sc_pool(tv, xt, length, b16):
    mesh = plsc.VectorSubcoreMesh(core_axis_name="c", subcore_axis_name="s")

    @functools.partial(
        pl.kernel,
        mesh=mesh,
        out_type=jax.ShapeDtypeStruct((_B,), jnp.float32),
        scratch_types=[
            pltpu.VMEM((_IDXN,), jnp.int32),
            pltpu.VMEM((_IDXN,), jnp.float32),
            pltpu.VMEM((_RPW,), jnp.float32),
            pltpu.VMEM((_RPW,), jnp.float32),
            pltpu.VMEM((16,), jnp.float32),
            pltpu.SemaphoreType.DMA,
            pltpu.SemaphoreType.DMA,
        ],
    )
    def sck(tv_hbm, xt_hbm, len_hbm, b_hbm, out_hbm,
            idx_v, val_v, len_v, out_v, b_v, sem, sem2):
        w = lax.axis_index("s") * 2 + lax.axis_index("c")
        row0 = w * _RPW
        pltpu.sync_copy(len_hbm.at[pl.ds(row0, _RPW)], len_v)
        pltpu.sync_copy(b_hbm, b_v)
        bias = b_v[...]
        for c in range(_NCH):
            col0 = row0 + c * _CH

            def stage(jb, _, _col0=col0):
                hs = [pltpu.async_copy(
                          xt_hbm.at[jb * 25 + k, pl.ds(_col0, _CH)],
                          idx_v.at[pl.ds((jb * 25 + k) * _CH, _CH)], sem2)
                      for k in range(25)]
                for h in hs:
                    h.wait()
                return 0

            lax.fori_loop(0, _L // 25, stage, 0)
            pltpu.async_copy(tv_hbm.at[idx_v], val_v, sem).wait()

            def jbody(j, accs):
                return tuple(
                    accs[g] + val_v[pl.ds(j * _CH + g * 16, 16)]
                    for g in range(_CH // 16))

            accs = lax.fori_loop(
                0, _L, jbody,
                tuple(jnp.zeros((16,), jnp.float32)
                      for _ in range(_CH // 16)))
            for g in range(_CH // 16):
                o16 = c * _CH + g * 16
                out_v[pl.ds(o16, 16)] = (accs[g] / len_v[pl.ds(o16, 16)]
                                         + bias)
        pltpu.sync_copy(out_v, out_hbm.at[pl.ds(row0, _RPW)])

    return sck(tv, xt, length, b16)


def kernel(x, length, table, W, b):
    tv = _table_times_w(table.T, W)
    b16 = jnp.broadcast_to(b.astype(jnp.float32), (16,))
    rows = _sc_pool(tv, x.T, length, b16)
    return rows.reshape(_B, 1)


# speedup vs baseline: 1.7376x; 1.0518x over previous
import functools

import jax
import jax.numpy as jnp
from jax import lax
from jax.experimental import pallas as pl
from jax.experimental.pallas import tpu as pltpu
from jax.experimental.pallas import tpu_sc as plsc

_VOCAB = 1000000
_EMB = 64
_B = 16384
_L = 200

_TBLK = 24576

_NW = 32
_RPW = _B // _NW
_CH = 128
_NCH = _RPW // _CH
_IDXN = _CH * _L


def _tv_body(t_ref, w_ref, o_ref):
    o_ref[...] = jnp.sum(t_ref[...] * w_ref[...], axis=0)


def _table_times_w(table_t, W):
    return pl.pallas_call(
        _tv_body,
        grid=(pl.cdiv(_VOCAB, _TBLK),),
        in_specs=[
            pl.BlockSpec((_EMB, _TBLK), lambda i: (0, i)),
            pl.BlockSpec((_EMB, 1), lambda i: (0, 0)),
        ],
        out_specs=pl.BlockSpec((_TBLK,), lambda i: (i,)),
        out_shape=jax.ShapeDtypeStruct((_VOCAB,), jnp.float32),
    )(table_t, W)


def _sc_stage(xt):
    mesh = plsc.VectorSubcoreMesh(core_axis_name="c", subcore_axis_name="s")

    @functools.partial(
        pl.kernel,
        mesh=mesh,
        out_type=jax.ShapeDtypeStruct((_B * _L,), jnp.int32),
        scratch_types=[
            pltpu.VMEM((_IDXN,), jnp.int32),
            pltpu.SemaphoreType.DMA,
        ],
    )
    def stg(xt_hbm, xs_hbm, idx_v, sem):
        w = lax.axis_index("s") * 2 + lax.axis_index("c")
        row0 = w * _RPW
        for c in range(_NCH):
            col0 = row0 + c * _CH

            def stage(jb, _, _col0=col0):
                hs = [pltpu.async_copy(
                          xt_hbm.at[jb * 25 + k, pl.ds(_col0, _CH)],
                          idx_v.at[pl.ds((jb * 25 + k) * _CH, _CH)], sem)
                      for k in range(25)]
                for h in hs:
                    h.wait()
                return 0

            lax.fori_loop(0, _L // 25, stage, 0)
            pltpu.sync_copy(idx_v, xs_hbm.at[pl.ds(col0 * _L, _IDXN)])

    return stg(xt)


def _sc_pool(tv, xs, length, b16):
    mesh = plsc.VectorSubcoreMesh(core_axis_name="c", subcore_axis_name="s")

    @functools.partial(
        pl.kernel,
        mesh=mesh,
        out_type=jax.ShapeDtypeStruct((_B,), jnp.float32),
        scratch_types=[
            pltpu.VMEM((_IDXN,), jnp.int32),
            pltpu.VMEM((_IDXN,), jnp.int32),
            pltpu.VMEM((_IDXN,), jnp.float32),
            pltpu.VMEM((_IDXN,), jnp.float32),
            pltpu.VMEM((_RPW,), jnp.float32),
            pltpu.VMEM((_RPW,), jnp.float32),
            pltpu.VMEM((16,), jnp.float32),
            pltpu.SemaphoreType.DMA,
            pltpu.SemaphoreType.DMA,
        ],
    )
    def sck(tv_hbm, xs_hbm, len_hbm, b_hbm, out_hbm,
            idx0, idx1, val0, val1, len_v, out_v, b_v, s0, s1):
        w = lax.axis_index("s") * 2 + lax.axis_index("c")
        row0 = w * _RPW
        pltpu.sync_copy(len_hbm.at[pl.ds(row0, _RPW)], len_v)
        pltpu.sync_copy(b_hbm, b_v)
        bias = b_v[...]
        bufs = [(idx0, val0, s0), (idx1, val1, s1)]
        pltpu.sync_copy(xs_hbm.at[pl.ds(row0 * _L, _IDXN)], idx0)
        handles = {0: pltpu.async_copy(tv_hbm.at[idx0], val0, s0)}
        for c in range(_NCH):
            _, val_v, _ = bufs[c % 2]
            if c + 1 < _NCH:
                nidx, nval, nsem = bufs[(c + 1) % 2]
                pltpu.sync_copy(
                    xs_hbm.at[pl.ds((row0 + (c + 1) * _CH) * _L, _IDXN)],
                    nidx)
                handles[c + 1] = pltpu.async_copy(
                    tv_hbm.at[nidx], nval, nsem)
            handles[c].wait()

            def jbody(j, accs, _val_v=val_v):
                return tuple(
                    accs[g] + _val_v[pl.ds(j * _CH + g * 16, 16)]
                    for g in range(_CH // 16))

            accs = lax.fori_loop(
                0, _L, jbody,
                tuple(jnp.zeros((16,), jnp.float32)
                      for _ in range(_CH // 16)))
            for g in range(_CH // 16):
                o16 = c * _CH + g * 16
                out_v[pl.ds(o16, 16)] = (accs[g] / len_v[pl.ds(o16, 16)]
                                         + bias)
        pltpu.sync_copy(out_v, out_hbm.at[pl.ds(row0, _RPW)])

    return sck(tv, xs, length, b16)


def kernel(x, length, table, W, b):
    xs = _sc_stage(x.T)
    tv = _table_times_w(table.T, W)
    b16 = jnp.broadcast_to(b.astype(jnp.float32), (16,))
    rows = _sc_pool(tv, xs, length, b16)
    return rows.reshape(_B, 1)


# speedup vs baseline: 1.7650x; 1.0158x over previous
import functools

import jax
import jax.numpy as jnp
from jax import lax
from jax.experimental import pallas as pl
from jax.experimental.pallas import tpu as pltpu
from jax.experimental.pallas import tpu_sc as plsc

_VOCAB = 1000000
_EMB = 64
_B = 16384
_L = 200

_TBLK = 49152

_NW = 32
_RPW = _B // _NW
_CH = 128
_NCH = _RPW // _CH
_IDXN = _CH * _L


def _tv_body(t_ref, w_ref, o_ref):
    o_ref[...] = jnp.sum(t_ref[...] * w_ref[...], axis=0)


def _table_times_w(table_t, W):
    return pl.pallas_call(
        _tv_body,
        grid=(pl.cdiv(_VOCAB, _TBLK),),
        in_specs=[
            pl.BlockSpec((_EMB, _TBLK), lambda i: (0, i)),
            pl.BlockSpec((_EMB, 1), lambda i: (0, 0)),
        ],
        out_specs=pl.BlockSpec((_TBLK,), lambda i: (i,)),
        out_shape=jax.ShapeDtypeStruct((_VOCAB,), jnp.float32),
    )(table_t, W)


def _sc_stage(xt):
    mesh = plsc.VectorSubcoreMesh(core_axis_name="c", subcore_axis_name="s")

    @functools.partial(
        pl.kernel,
        mesh=mesh,
        out_type=jax.ShapeDtypeStruct((_B * _L,), jnp.int32),
        scratch_types=[
            pltpu.VMEM((_IDXN,), jnp.int32),
            pltpu.SemaphoreType.DMA,
        ],
    )
    def stg(xt_hbm, xs_hbm, idx_v, sem):
        w = lax.axis_index("s") * 2 + lax.axis_index("c")
        row0 = w * _RPW
        for c in range(_NCH):
            col0 = row0 + c * _CH

            def stage(jb, _, _col0=col0):
                hs = [pltpu.async_copy(
                          xt_hbm.at[jb * 25 + k, pl.ds(_col0, _CH)],
                          idx_v.at[pl.ds((jb * 25 + k) * _CH, _CH)], sem)
                      for k in range(25)]
                for h in hs:
                    h.wait()
                return 0

            lax.fori_loop(0, _L // 25, stage, 0)
            pltpu.sync_copy(idx_v, xs_hbm.at[pl.ds(col0 * _L, _IDXN)])

    return stg(xt)


def _sc_pool(tv, xs, length, b16):
    mesh = plsc.VectorSubcoreMesh(core_axis_name="c", subcore_axis_name="s")

    @functools.partial(
        pl.kernel,
        mesh=mesh,
        out_type=jax.ShapeDtypeStruct((_B,), jnp.float32),
        scratch_types=[
            pltpu.VMEM((_IDXN,), jnp.int32),
            pltpu.VMEM((_IDXN,), jnp.int32),
            pltpu.VMEM((_IDXN,), jnp.float32),
            pltpu.VMEM((_IDXN,), jnp.float32),
            pltpu.VMEM((_RPW,), jnp.float32),
            pltpu.VMEM((_RPW,), jnp.float32),
            pltpu.VMEM((16,), jnp.float32),
            pltpu.SemaphoreType.DMA,
            pltpu.SemaphoreType.DMA,
        ],
    )
    def sck(tv_hbm, xs_hbm, len_hbm, b_hbm, out_hbm,
            idx0, idx1, val0, val1, len_v, out_v, b_v, s0, s1):
        w = lax.axis_index("s") * 2 + lax.axis_index("c")
        row0 = w * _RPW
        bufs = [(idx0, val0, s0), (idx1, val1, s1)]
        pltpu.sync_copy(xs_hbm.at[pl.ds(row0 * _L, _IDXN)], idx0)
        handles = {0: pltpu.async_copy(tv_hbm.at[idx0], val0, s0)}
        pltpu.sync_copy(len_hbm.at[pl.ds(row0, _RPW)], len_v)
        pltpu.sync_copy(b_hbm, b_v)
        bias = b_v[...]
        for c in range(_NCH):
            _, val_v, _ = bufs[c % 2]
            if c + 1 < _NCH:
                nidx, nval, nsem = bufs[(c + 1) % 2]
                pltpu.sync_copy(
                    xs_hbm.at[pl.ds((row0 + (c + 1) * _CH) * _L, _IDXN)],
                    nidx)
                handles[c + 1] = pltpu.async_copy(
                    tv_hbm.at[nidx], nval, nsem)
            handles[c].wait()

            def jbody(j, accs, _val_v=val_v):
                return tuple(
                    accs[g] + _val_v[pl.ds(j * _CH + g * 16, 16)]
                    for g in range(_CH // 16))

            accs = lax.fori_loop(
                0, _L, jbody,
                tuple(jnp.zeros((16,), jnp.float32)
                      for _ in range(_CH // 16)))
            for g in range(_CH // 16):
                o16 = c * _CH + g * 16
                out_v[pl.ds(o16, 16)] = (accs[g] / len_v[pl.ds(o16, 16)]
                                         + bias)
        pltpu.sync_copy(out_v, out_hbm.at[pl.ds(row0, _RPW)])

    return sck(tv, xs, length, b16)


def kernel(x, length, table, W, b):
    xs = _sc_stage(x.T)
    tv = _table_times_w(table.T, W)
    b16 = jnp.broadcast_to(b.astype(jnp.float32), (16,))
    rows = _sc_pool(tv, xs, length, b16)
    return rows.reshape(_B, 1)
